# Initial kernel scaffold; baseline (speedup 1.0000x reference)
import functools

import numpy as np
import jax
import jax.numpy as jnp
from jax import lax
from jax.experimental import pallas as pl
from jax.experimental.pallas import tpu as pltpu
from jax.experimental.pallas import tpu_sc as plsc

R = 128
C = 32768
L = 16
NC = 2
NS = 16
NW = NC * NS
ROWS_PER_W = R // NW

_PERM_CACHE = None


def _perm_host():
    global _PERM_CACHE
    if _PERM_CACHE is None:
        k = jax.random.fold_in(jax.random.key(42), 0)
        _PERM_CACHE = np.asarray(jax.random.permutation(k, C)).astype(np.int32)
    return _PERM_CACHE


_mesh = plsc.VectorSubcoreMesh(core_axis_name="c", subcore_axis_name="s")


@functools.partial(
    pl.kernel,
    out_type=jax.ShapeDtypeStruct((R, C), jnp.float32),
    mesh=_mesh,
    scratch_types=[
        pltpu.VMEM((C,), jnp.int32),
        pltpu.VMEM((C,), jnp.float32),
        pltpu.VMEM((C,), jnp.float32),
    ],
)
def _shuffle_rows(x_hbm, perm_hbm, out_hbm, perm_v, rin_v, rout_v):
    wid = lax.axis_index("s") * NC + lax.axis_index("c")
    pltpu.sync_copy(perm_hbm, perm_v)
    for rr in range(ROWS_PER_W):
        r = wid * ROWS_PER_W + rr
        pltpu.sync_copy(x_hbm.at[r], rin_v)

        @plsc.parallel_loop(0, C, step=L, unroll=8)
        def _(i):
            idx = perm_v[pl.ds(i, L)]
            rout_v[pl.ds(i, L)] = plsc.load_gather(rin_v, [idx])

        pltpu.sync_copy(rout_v, out_hbm.at[r])


def kernel(x):
    perm = jnp.asarray(_perm_host())
    return _shuffle_rows(x, perm)


# speedup vs baseline: 1.2270x; 1.2270x over previous
import functools

import numpy as np
import jax
import jax.numpy as jnp
from jax import lax
from jax.experimental import pallas as pl
from jax.experimental.pallas import tpu as pltpu
from jax.experimental.pallas import tpu_sc as plsc

R = 128
C = 32768
L = 16
NC = 2
NS = 16
NW = NC * NS
ROWS_PER_W = R // NW


_mesh = plsc.VectorSubcoreMesh(core_axis_name="c", subcore_axis_name="s")


@functools.partial(
    pl.kernel,
    out_type=jax.ShapeDtypeStruct((R, C), jnp.float32),
    mesh=_mesh,
    compiler_params=pltpu.CompilerParams(needs_layout_passes=False),
    scratch_types=[
        pltpu.VMEM((C,), jnp.int32),
        pltpu.VMEM((C,), jnp.float32),
        pltpu.VMEM((C,), jnp.float32),
    ],
)
def _shuffle_rows(x_hbm, perm_hbm, out_hbm, perm_v, rin_v, rout_v):
    wid = lax.axis_index("s") * NC + lax.axis_index("c")
    pltpu.sync_copy(perm_hbm, perm_v)
    for rr in range(ROWS_PER_W):
        r = wid * ROWS_PER_W + rr
        pltpu.sync_copy(x_hbm.at[r], rin_v)

        @plsc.parallel_loop(0, C, step=L, unroll=8)
        def _(i):
            idx = perm_v[pl.ds(i, L)]
            rout_v[pl.ds(i, L)] = plsc.load_gather(rin_v, [idx])

        pltpu.sync_copy(rout_v, out_hbm.at[r])


def kernel(x):
    k = jax.random.fold_in(jax.random.key(42), 0)
    perm = jax.random.permutation(k, C).astype(jnp.int32)
    return _shuffle_rows(x, perm)


# speedup vs baseline: 2.3507x; 1.9159x over previous
import functools

import numpy as np
import jax
import jax.numpy as jnp
from jax import lax
from jax.experimental import pallas as pl
from jax.experimental.pallas import tpu as pltpu
from jax.experimental.pallas import tpu_sc as plsc

R = 128
C = 32768
L = 16
NC = 2
NS = 16
NW = NC * NS
ROWS_PER_W = R // NW

_U32 = np.uint32


def _rotl(x, d):
    x = x.astype(np.uint32)
    return ((x << _U32(d)) | (x >> _U32(32 - d))).astype(np.uint32)


def _threefry2x32(k1, k2, x1, x2):
    k1 = _U32(k1)
    k2 = _U32(k2)
    ks = [k1, k2, _U32(k1 ^ k2 ^ _U32(0x1BD11BDA))]
    rot = [[13, 15, 26, 6], [17, 29, 16, 24]]
    x = np.asarray(x1, np.uint32) + ks[0]
    y = np.asarray(x2, np.uint32) + ks[1]

    def rounds(x, y, rs):
        for r in rs:
            x = (x + y).astype(np.uint32)
            y = _rotl(y, r)
            y = x ^ y
        return x, y

    for i, (rs, kx, kofs) in enumerate(
        [(rot[0], 1, 2), (rot[1], 2, 0), (rot[0], 0, 1),
         (rot[1], 1, 2), (rot[0], 2, 0)]
    ):
        x, y = rounds(x, y, rs)
        x = (x + ks[kx]).astype(np.uint32)
        y = (y + ks[kofs] + _U32(i + 1)).astype(np.uint32)
    return x, y


def _fixed_permutation(n=C, seed=42):
    a, b = _threefry2x32(_U32(0), _U32(seed), np.array([0], np.uint32),
                         np.array([0], np.uint32))
    key = (a[0], b[0])

    perm = np.arange(n, dtype=np.int32)
    num_rounds = int(np.ceil(3 * np.log(n) / np.log(2**32 - 1)))
    for _ in range(num_rounds):
        b1, b2 = _threefry2x32(key[0], key[1], np.zeros(2, np.uint32),
                               np.arange(2, dtype=np.uint32))
        key = (b1[0], b2[0])
        subkey = (b1[1], b2[1])
        c1, c2 = _threefry2x32(subkey[0], subkey[1], np.zeros(n, np.uint32),
                               np.arange(n, dtype=np.uint32))
        perm = perm[np.argsort((c1 ^ c2).astype(np.uint32), kind="stable")]
    return perm


_PERM_HOST = _fixed_permutation()

@functools.cache
def _build_shuffle():
    mesh = plsc.VectorSubcoreMesh(
        core_axis_name="c", subcore_axis_name="s", num_cores=NC, num_subcores=NS
    )

    @functools.partial(
        pl.kernel,
        out_type=jax.ShapeDtypeStruct((R, C), jnp.float32),
        mesh=mesh,
        compiler_params=pltpu.CompilerParams(needs_layout_passes=False),
        scratch_types=[
            pltpu.VMEM((C,), jnp.int32),
            pltpu.VMEM((C,), jnp.float32),
            pltpu.VMEM((C,), jnp.float32),
        ],
    )
    def _shuffle_rows(x_hbm, perm_hbm, out_hbm, perm_v, rin_v, rout_v):
        wid = lax.axis_index("s") * NC + lax.axis_index("c")
        pltpu.sync_copy(perm_hbm, perm_v)
        for rr in range(ROWS_PER_W):
            r = wid * ROWS_PER_W + rr
            pltpu.sync_copy(x_hbm.at[r], rin_v)

            @plsc.parallel_loop(0, C, step=L, unroll=8)
            def _(i):
                idx = perm_v[pl.ds(i, L)]
                rout_v[pl.ds(i, L)] = plsc.load_gather(rin_v, [idx])

            pltpu.sync_copy(rout_v, out_hbm.at[r])

    return _shuffle_rows


def kernel(x):
    perm = jnp.asarray(_PERM_HOST)
    return _build_shuffle()(x, perm)


# speedup vs baseline: 2.8246x; 1.2016x over previous
import functools

import numpy as np
import jax
import jax.numpy as jnp
from jax import lax
from jax.experimental import pallas as pl
from jax.experimental.pallas import tpu as pltpu
from jax.experimental.pallas import tpu_sc as plsc

R = 128
C = 32768
L = 16
NC = 2
NS = 16
NW = NC * NS
ROWS_PER_W = R // NW

_U32 = np.uint32


def _rotl(x, d):
    x = x.astype(np.uint32)
    return ((x << _U32(d)) | (x >> _U32(32 - d))).astype(np.uint32)


def _threefry2x32(k1, k2, x1, x2):
    k1 = _U32(k1)
    k2 = _U32(k2)
    ks = [k1, k2, _U32(k1 ^ k2 ^ _U32(0x1BD11BDA))]
    rot = [[13, 15, 26, 6], [17, 29, 16, 24]]
    x = np.asarray(x1, np.uint32) + ks[0]
    y = np.asarray(x2, np.uint32) + ks[1]

    def rounds(x, y, rs):
        for r in rs:
            x = (x + y).astype(np.uint32)
            y = _rotl(y, r)
            y = x ^ y
        return x, y

    for i, (rs, kx, kofs) in enumerate(
        [(rot[0], 1, 2), (rot[1], 2, 0), (rot[0], 0, 1),
         (rot[1], 1, 2), (rot[0], 2, 0)]
    ):
        x, y = rounds(x, y, rs)
        x = (x + ks[kx]).astype(np.uint32)
        y = (y + ks[kofs] + _U32(i + 1)).astype(np.uint32)
    return x, y


def _fixed_permutation(n=C, seed=42):
    a, b = _threefry2x32(_U32(0), _U32(seed), np.array([0], np.uint32),
                         np.array([0], np.uint32))
    key = (a[0], b[0])

    perm = np.arange(n, dtype=np.int32)
    num_rounds = int(np.ceil(3 * np.log(n) / np.log(2**32 - 1)))
    for _ in range(num_rounds):
        b1, b2 = _threefry2x32(key[0], key[1], np.zeros(2, np.uint32),
                               np.arange(2, dtype=np.uint32))
        key = (b1[0], b2[0])
        subkey = (b1[1], b2[1])
        c1, c2 = _threefry2x32(subkey[0], subkey[1], np.zeros(n, np.uint32),
                               np.arange(n, dtype=np.uint32))
        perm = perm[np.argsort((c1 ^ c2).astype(np.uint32), kind="stable")]
    return perm


_PERM_HOST = _fixed_permutation()

OCH = 4096
NCHUNK = C // OCH


@functools.cache
def _build_shuffle():
    mesh = plsc.VectorSubcoreMesh(
        core_axis_name="c", subcore_axis_name="s", num_cores=NC, num_subcores=NS
    )

    @functools.partial(
        pl.kernel,
        out_type=jax.ShapeDtypeStruct((R, C), jnp.float32),
        mesh=mesh,
        compiler_params=pltpu.CompilerParams(needs_layout_passes=False),
        scratch_types=[
            pltpu.VMEM((C,), jnp.int32),
            pltpu.VMEM((C,), jnp.float32),
            pltpu.VMEM((C,), jnp.float32),
            pltpu.VMEM((OCH,), jnp.float32),
            pltpu.VMEM((OCH,), jnp.float32),
            pltpu.SemaphoreType.DMA,
            pltpu.SemaphoreType.DMA,
            pltpu.SemaphoreType.DMA,
            pltpu.SemaphoreType.DMA,
            pltpu.SemaphoreType.DMA,
        ],
    )
    def _shuffle_rows(x_hbm, perm_hbm, out_hbm, perm_v, rin0, rin1, ob0, ob1,
                      sem_p, si0, si1, so0, so1):
        wid = lax.axis_index("s") * NC + lax.axis_index("c")
        r0 = wid * ROWS_PER_W
        rins = [rin0, rin1]
        in_sems = [si0, si1]
        obufs = [ob0, ob1]
        out_sems = [so0, so1]

        cp_perm = pltpu.async_copy(perm_hbm, perm_v, sem_p)
        cp_in = [pltpu.async_copy(x_hbm.at[r0], rin0, si0), None]
        cp_perm.wait()

        out_pending = [None, None]
        g = 0
        for rr in range(ROWS_PER_W):
            cur = rr % 2
            cp_in[cur].wait()
            if rr + 1 < ROWS_PER_W:
                nxt = (rr + 1) % 2
                cp_in[nxt] = pltpu.async_copy(
                    x_hbm.at[r0 + rr + 1], rins[nxt], in_sems[nxt])
            for k in range(NCHUNK):
                slot = g % 2
                if out_pending[slot] is not None:
                    out_pending[slot].wait()
                base = k * OCH

                @plsc.parallel_loop(0, OCH, step=L, unroll=8)
                def _(i, _base=base, _cur=cur, _slot=slot):
                    idx = perm_v[pl.ds(_base + i, L)]
                    obufs[_slot][pl.ds(i, L)] = plsc.load_gather(
                        rins[_cur], [idx])

                out_pending[slot] = pltpu.async_copy(
                    obufs[slot], out_hbm.at[r0 + rr, pl.ds(base, OCH)],
                    out_sems[slot])
                g += 1
        for slot in (0, 1):
            if out_pending[slot] is not None:
                out_pending[slot].wait()

    return _shuffle_rows


def kernel(x):
    perm = jnp.asarray(_PERM_HOST)
    return _build_shuffle()(x, perm)


# speedup vs baseline: 2.9062x; 1.0289x over previous
import functools

import numpy as np
import jax
import jax.numpy as jnp
from jax import lax
from jax.experimental import pallas as pl
from jax.experimental.pallas import tpu as pltpu
from jax.experimental.pallas import tpu_sc as plsc

R = 128
C = 32768
L = 16
NC = 2
NS = 16
NW = NC * NS
ROWS_PER_W = R // NW

_U32 = np.uint32


def _rotl(x, d):
    x = x.astype(np.uint32)
    return ((x << _U32(d)) | (x >> _U32(32 - d))).astype(np.uint32)


def _threefry2x32(k1, k2, x1, x2):
    k1 = _U32(k1)
    k2 = _U32(k2)
    ks = [k1, k2, _U32(k1 ^ k2 ^ _U32(0x1BD11BDA))]
    rot = [[13, 15, 26, 6], [17, 29, 16, 24]]
    x = np.asarray(x1, np.uint32) + ks[0]
    y = np.asarray(x2, np.uint32) + ks[1]

    def rounds(x, y, rs):
        for r in rs:
            x = (x + y).astype(np.uint32)
            y = _rotl(y, r)
            y = x ^ y
        return x, y

    for i, (rs, kx, kofs) in enumerate(
        [(rot[0], 1, 2), (rot[1], 2, 0), (rot[0], 0, 1),
         (rot[1], 1, 2), (rot[0], 2, 0)]
    ):
        x, y = rounds(x, y, rs)
        x = (x + ks[kx]).astype(np.uint32)
        y = (y + ks[kofs] + _U32(i + 1)).astype(np.uint32)
    return x, y


def _fixed_permutation(n=C, seed=42):
    a, b = _threefry2x32(_U32(0), _U32(seed), np.array([0], np.uint32),
                         np.array([0], np.uint32))
    key = (a[0], b[0])

    perm = np.arange(n, dtype=np.int32)
    num_rounds = int(np.ceil(3 * np.log(n) / np.log(2**32 - 1)))
    for _ in range(num_rounds):
        b1, b2 = _threefry2x32(key[0], key[1], np.zeros(2, np.uint32),
                               np.arange(2, dtype=np.uint32))
        key = (b1[0], b2[0])
        subkey = (b1[1], b2[1])
        c1, c2 = _threefry2x32(subkey[0], subkey[1], np.zeros(n, np.uint32),
                               np.arange(n, dtype=np.uint32))
        perm = perm[np.argsort((c1 ^ c2).astype(np.uint32), kind="stable")]
    return perm


_PERM_HOST = _fixed_permutation()

OCH = 8192
NCHUNK = C // OCH


@functools.cache
def _build_shuffle():
    mesh = plsc.VectorSubcoreMesh(
        core_axis_name="c", subcore_axis_name="s", num_cores=NC, num_subcores=NS
    )

    @functools.partial(
        pl.kernel,
        out_type=jax.ShapeDtypeStruct((R, C), jnp.float32),
        mesh=mesh,
        compiler_params=pltpu.CompilerParams(needs_layout_passes=False),
        scratch_types=[
            pltpu.VMEM((C,), jnp.int32),
            pltpu.VMEM((C,), jnp.float32),
            pltpu.VMEM((C,), jnp.float32),
            pltpu.VMEM((OCH,), jnp.float32),
            pltpu.VMEM((OCH,), jnp.float32),
            pltpu.SemaphoreType.DMA,
            pltpu.SemaphoreType.DMA,
            pltpu.SemaphoreType.DMA,
            pltpu.SemaphoreType.DMA,
            pltpu.SemaphoreType.DMA,
        ],
    )
    def _shuffle_rows(x_hbm, perm_hbm, out_hbm, perm_v, rin0, rin1, ob0, ob1,
                      sem_p, si0, si1, so0, so1):
        wid = lax.axis_index("s") * NC + lax.axis_index("c")
        r0 = wid * ROWS_PER_W
        rins = [rin0, rin1]
        in_sems = [si0, si1]
        obufs = [ob0, ob1]
        out_sems = [so0, so1]

        cp_perm = pltpu.async_copy(perm_hbm, perm_v, sem_p)
        cp_in = [pltpu.async_copy(x_hbm.at[r0], rin0, si0), None]
        cp_perm.wait()

        out_pending = [None, None]
        g = 0
        for rr in range(ROWS_PER_W):
            cur = rr % 2
            cp_in[cur].wait()
            if rr + 1 < ROWS_PER_W:
                nxt = (rr + 1) % 2
                cp_in[nxt] = pltpu.async_copy(
                    x_hbm.at[r0 + rr + 1], rins[nxt], in_sems[nxt])
            for k in range(NCHUNK):
                slot = g % 2
                if out_pending[slot] is not None:
                    out_pending[slot].wait()
                base = k * OCH

                @plsc.parallel_loop(0, OCH, step=L, unroll=4)
                def _(i, _base=base, _cur=cur, _slot=slot):
                    idx = perm_v[pl.ds(_base + i, L)]
                    obufs[_slot][pl.ds(i, L)] = plsc.load_gather(
                        rins[_cur], [idx])

                out_pending[slot] = pltpu.async_copy(
                    obufs[slot], out_hbm.at[r0 + rr, pl.ds(base, OCH)],
                    out_sems[slot])
                g += 1
        for slot in (0, 1):
            if out_pending[slot] is not None:
                out_pending[slot].wait()

    return _shuffle_rows


def kernel(x):
    perm = jnp.asarray(_PERM_HOST)
    return _build_shuffle()(x, perm)


# speedup vs baseline: 2.9583x; 1.0179x over previous
import functools

import numpy as np
import jax
import jax.numpy as jnp
from jax import lax
from jax.experimental import pallas as pl
from jax.experimental.pallas import tpu as pltpu
from jax.experimental.pallas import tpu_sc as plsc

R = 128
C = 32768
L = 16
NC = 2
NS = 16
NW = NC * NS
ROWS_PER_W = R // NW

_U32 = np.uint32


def _rotl(x, d):
    x = x.astype(np.uint32)
    return ((x << _U32(d)) | (x >> _U32(32 - d))).astype(np.uint32)


def _threefry2x32(k1, k2, x1, x2):
    k1 = _U32(k1)
    k2 = _U32(k2)
    ks = [k1, k2, _U32(k1 ^ k2 ^ _U32(0x1BD11BDA))]
    rot = [[13, 15, 26, 6], [17, 29, 16, 24]]
    x = np.asarray(x1, np.uint32) + ks[0]
    y = np.asarray(x2, np.uint32) + ks[1]

    def rounds(x, y, rs):
        for r in rs:
            x = (x + y).astype(np.uint32)
            y = _rotl(y, r)
            y = x ^ y
        return x, y

    for i, (rs, kx, kofs) in enumerate(
        [(rot[0], 1, 2), (rot[1], 2, 0), (rot[0], 0, 1),
         (rot[1], 1, 2), (rot[0], 2, 0)]
    ):
        x, y = rounds(x, y, rs)
        x = (x + ks[kx]).astype(np.uint32)
        y = (y + ks[kofs] + _U32(i + 1)).astype(np.uint32)
    return x, y


def _fixed_permutation(n=C, seed=42):
    a, b = _threefry2x32(_U32(0), _U32(seed), np.array([0], np.uint32),
                         np.array([0], np.uint32))
    key = (a[0], b[0])

    perm = np.arange(n, dtype=np.int32)
    num_rounds = int(np.ceil(3 * np.log(n) / np.log(2**32 - 1)))
    for _ in range(num_rounds):
        b1, b2 = _threefry2x32(key[0], key[1], np.zeros(2, np.uint32),
                               np.arange(2, dtype=np.uint32))
        key = (b1[0], b2[0])
        subkey = (b1[1], b2[1])
        c1, c2 = _threefry2x32(subkey[0], subkey[1], np.zeros(n, np.uint32),
                               np.arange(n, dtype=np.uint32))
        perm = perm[np.argsort((c1 ^ c2).astype(np.uint32), kind="stable")]
    return perm


_PERM_HOST = _fixed_permutation()

OCH = 8192
NCHUNK = C // OCH


@functools.cache
def _build_shuffle():
    mesh = plsc.VectorSubcoreMesh(
        core_axis_name="c", subcore_axis_name="s", num_cores=NC, num_subcores=NS
    )

    @functools.partial(
        pl.kernel,
        out_type=jax.ShapeDtypeStruct((R, C), jnp.float32),
        mesh=mesh,
        compiler_params=pltpu.CompilerParams(needs_layout_passes=False),
        scratch_types=[
            pltpu.VMEM((C,), jnp.int32),
            pltpu.VMEM((C,), jnp.float32),
            pltpu.VMEM((C,), jnp.float32),
            pltpu.VMEM((OCH,), jnp.float32),
            pltpu.VMEM((OCH,), jnp.float32),
            pltpu.SemaphoreType.DMA,
            pltpu.SemaphoreType.DMA,
            pltpu.SemaphoreType.DMA,
            pltpu.SemaphoreType.DMA,
            pltpu.SemaphoreType.DMA,
        ],
    )
    def _shuffle_rows(x_hbm, perm_hbm, out_hbm, perm_v, rin0, rin1, ob0, ob1,
                      sem_p, si0, si1, so0, so1):
        wid = lax.axis_index("s") * NC + lax.axis_index("c")
        r0 = wid * ROWS_PER_W
        rins = [rin0, rin1]
        in_sems = [si0, si1]
        obufs = [ob0, ob1]
        out_sems = [so0, so1]

        cp_perm = pltpu.async_copy(perm_hbm, perm_v, sem_p)
        cp_in = [pltpu.async_copy(x_hbm.at[r0], rin0, si0), None]
        cp_perm.wait()

        out_pending = [None, None]
        g = 0
        for rr in range(ROWS_PER_W):
            cur = rr % 2
            cp_in[cur].wait()
            if rr + 1 < ROWS_PER_W:
                nxt = (rr + 1) % 2
                cp_in[nxt] = pltpu.async_copy(
                    x_hbm.at[r0 + rr + 1], rins[nxt], in_sems[nxt])
            for k in range(NCHUNK):
                slot = g % 2
                if out_pending[slot] is not None:
                    out_pending[slot].wait()
                base = k * OCH

                @plsc.parallel_loop(0, OCH, step=L, unroll=8)
                def _(i, _base=base, _cur=cur, _slot=slot):
                    idx = perm_v[pl.ds(_base + i, L)]
                    obufs[_slot][pl.ds(i, L)] = plsc.load_gather(
                        rins[_cur], [idx])

                out_pending[slot] = pltpu.async_copy(
                    obufs[slot], out_hbm.at[r0 + rr, pl.ds(base, OCH)],
                    out_sems[slot])
                g += 1
        for slot in (0, 1):
            if out_pending[slot] is not None:
                out_pending[slot].wait()

    return _shuffle_rows


def kernel(x):
    perm = jnp.asarray(_PERM_HOST)
    return _build_shuffle()(x, perm)


# speedup vs baseline: 3.1091x; 1.0510x over previous
import functools

import numpy as np
import jax
import jax.numpy as jnp
from jax import lax
from jax.experimental import pallas as pl
from jax.experimental.pallas import tpu as pltpu
from jax.experimental.pallas import tpu_sc as plsc

R = 128
C = 32768
L = 16
NC = 2
NS = 16
NW = NC * NS
ROWS_PER_W = R // NW

_U32 = np.uint32


def _rotl(x, d):
    x = x.astype(np.uint32)
    return ((x << _U32(d)) | (x >> _U32(32 - d))).astype(np.uint32)


def _threefry2x32(k1, k2, x1, x2):
    k1 = _U32(k1)
    k2 = _U32(k2)
    ks = [k1, k2, _U32(k1 ^ k2 ^ _U32(0x1BD11BDA))]
    rot = [[13, 15, 26, 6], [17, 29, 16, 24]]
    x = np.asarray(x1, np.uint32) + ks[0]
    y = np.asarray(x2, np.uint32) + ks[1]

    def rounds(x, y, rs):
        for r in rs:
            x = (x + y).astype(np.uint32)
            y = _rotl(y, r)
            y = x ^ y
        return x, y

    for i, (rs, kx, kofs) in enumerate(
        [(rot[0], 1, 2), (rot[1], 2, 0), (rot[0], 0, 1),
         (rot[1], 1, 2), (rot[0], 2, 0)]
    ):
        x, y = rounds(x, y, rs)
        x = (x + ks[kx]).astype(np.uint32)
        y = (y + ks[kofs] + _U32(i + 1)).astype(np.uint32)
    return x, y


def _fixed_permutation(n=C, seed=42):
    a, b = _threefry2x32(_U32(0), _U32(seed), np.array([0], np.uint32),
                         np.array([0], np.uint32))
    key = (a[0], b[0])

    perm = np.arange(n, dtype=np.int32)
    num_rounds = int(np.ceil(3 * np.log(n) / np.log(2**32 - 1)))
    for _ in range(num_rounds):
        b1, b2 = _threefry2x32(key[0], key[1], np.zeros(2, np.uint32),
                               np.arange(2, dtype=np.uint32))
        key = (b1[0], b2[0])
        subkey = (b1[1], b2[1])
        c1, c2 = _threefry2x32(subkey[0], subkey[1], np.zeros(n, np.uint32),
                               np.arange(n, dtype=np.uint32))
        perm = perm[np.argsort((c1 ^ c2).astype(np.uint32), kind="stable")]
    return perm


_PERM_HOST = _fixed_permutation()

_PERM_PACKED_HOST = np.ascontiguousarray(
    _PERM_HOST.reshape(C // 32, 2, 16)[:, 0, :]
    | (_PERM_HOST.reshape(C // 32, 2, 16)[:, 1, :] << 16)
).reshape(C // 2).astype(np.int32)

OCH = 8192
NCHUNK = C // OCH


@functools.cache
def _build_shuffle():
    mesh = plsc.VectorSubcoreMesh(
        core_axis_name="c", subcore_axis_name="s", num_cores=NC, num_subcores=NS
    )

    @functools.partial(
        pl.kernel,
        out_type=jax.ShapeDtypeStruct((R, C), jnp.float32),
        mesh=mesh,
        compiler_params=pltpu.CompilerParams(needs_layout_passes=False),
        scratch_types=[
            pltpu.VMEM((C // 2,), jnp.int32),
            pltpu.VMEM((C,), jnp.float32),
            pltpu.VMEM((C,), jnp.float32),
            pltpu.VMEM((OCH,), jnp.float32),
            pltpu.VMEM((OCH,), jnp.float32),
            pltpu.SemaphoreType.DMA,
            pltpu.SemaphoreType.DMA,
            pltpu.SemaphoreType.DMA,
            pltpu.SemaphoreType.DMA,
            pltpu.SemaphoreType.DMA,
        ],
    )
    def _shuffle_rows(x_hbm, perm_hbm, out_hbm, perm_v, rin0, rin1, ob0, ob1,
                      sem_p, si0, si1, so0, so1):
        wid = lax.axis_index("s") * NC + lax.axis_index("c")
        r0 = wid * ROWS_PER_W
        rins = [rin0, rin1]
        in_sems = [si0, si1]
        obufs = [ob0, ob1]
        out_sems = [so0, so1]

        cp_perm = pltpu.async_copy(perm_hbm, perm_v, sem_p)
        cp_in = [pltpu.async_copy(x_hbm.at[r0], rin0, si0), None]
        cp_perm.wait()

        out_pending = [None, None]
        g = 0
        for rr in range(ROWS_PER_W):
            cur = rr % 2
            cp_in[cur].wait()
            if rr + 1 < ROWS_PER_W:
                nxt = (rr + 1) % 2
                cp_in[nxt] = pltpu.async_copy(
                    x_hbm.at[r0 + rr + 1], rins[nxt], in_sems[nxt])
            for k in range(NCHUNK):
                slot = g % 2
                if out_pending[slot] is not None:
                    out_pending[slot].wait()
                base = k * (OCH // 2)

                @plsc.parallel_loop(0, OCH // 2, step=L, unroll=4)
                def _(j, _base=base, _cur=cur, _slot=slot):
                    w = perm_v[pl.ds(_base + j, L)]
                    lo = lax.bitwise_and(w, jnp.int32(0xFFFF))
                    hi = lax.shift_right_logical(w, jnp.int32(16))
                    obufs[_slot][pl.ds(2 * j, L)] = plsc.load_gather(
                        rins[_cur], [lo])
                    obufs[_slot][pl.ds(2 * j + L, L)] = plsc.load_gather(
                        rins[_cur], [hi])

                out_pending[slot] = pltpu.async_copy(
                    obufs[slot], out_hbm.at[r0 + rr, pl.ds(k * OCH, OCH)],
                    out_sems[slot])
                g += 1
        for slot in (0, 1):
            if out_pending[slot] is not None:
                out_pending[slot].wait()

    return _shuffle_rows


def kernel(x):
    perm = jnp.asarray(_PERM_PACKED_HOST)
    return _build_shuffle()(x, perm)


# speedup vs baseline: 3.1210x; 1.0038x over previous
import functools

import numpy as np
import jax
import jax.numpy as jnp
from jax import lax
from jax.experimental import pallas as pl
from jax.experimental.pallas import tpu as pltpu
from jax.experimental.pallas import tpu_sc as plsc

R = 128
C = 32768
L = 16
NC = 2
NS = 16
NW = NC * NS
ROWS_PER_W = R // NW

_U32 = np.uint32


def _rotl(x, d):
    x = x.astype(np.uint32)
    return ((x << _U32(d)) | (x >> _U32(32 - d))).astype(np.uint32)


def _threefry2x32(k1, k2, x1, x2):
    k1 = _U32(k1)
    k2 = _U32(k2)
    ks = [k1, k2, _U32(k1 ^ k2 ^ _U32(0x1BD11BDA))]
    rot = [[13, 15, 26, 6], [17, 29, 16, 24]]
    x = np.asarray(x1, np.uint32) + ks[0]
    y = np.asarray(x2, np.uint32) + ks[1]

    def rounds(x, y, rs):
        for r in rs:
            x = (x + y).astype(np.uint32)
            y = _rotl(y, r)
            y = x ^ y
        return x, y

    for i, (rs, kx, kofs) in enumerate(
        [(rot[0], 1, 2), (rot[1], 2, 0), (rot[0], 0, 1),
         (rot[1], 1, 2), (rot[0], 2, 0)]
    ):
        x, y = rounds(x, y, rs)
        x = (x + ks[kx]).astype(np.uint32)
        y = (y + ks[kofs] + _U32(i + 1)).astype(np.uint32)
    return x, y


def _fixed_permutation(n=C, seed=42):
    a, b = _threefry2x32(_U32(0), _U32(seed), np.array([0], np.uint32),
                         np.array([0], np.uint32))
    key = (a[0], b[0])

    perm = np.arange(n, dtype=np.int32)
    num_rounds = int(np.ceil(3 * np.log(n) / np.log(2**32 - 1)))
    for _ in range(num_rounds):
        b1, b2 = _threefry2x32(key[0], key[1], np.zeros(2, np.uint32),
                               np.arange(2, dtype=np.uint32))
        key = (b1[0], b2[0])
        subkey = (b1[1], b2[1])
        c1, c2 = _threefry2x32(subkey[0], subkey[1], np.zeros(n, np.uint32),
                               np.arange(n, dtype=np.uint32))
        perm = perm[np.argsort((c1 ^ c2).astype(np.uint32), kind="stable")]
    return perm


_PERM_HOST = _fixed_permutation()

_PERM_PACKED_HOST = np.ascontiguousarray(
    _PERM_HOST.reshape(C // 32, 2, 16)[:, 0, :]
    | (_PERM_HOST.reshape(C // 32, 2, 16)[:, 1, :] << 16)
).reshape(C // 2).astype(np.int32)

OCH = 8192
NCHUNK = C // OCH


@functools.cache
def _build_shuffle():
    mesh = plsc.VectorSubcoreMesh(
        core_axis_name="c", subcore_axis_name="s", num_cores=NC, num_subcores=NS
    )

    @functools.partial(
        pl.kernel,
        out_type=jax.ShapeDtypeStruct((R, C), jnp.float32),
        mesh=mesh,
        compiler_params=pltpu.CompilerParams(
            needs_layout_passes=False,
            disable_bounds_checks=True,
            disable_semaphore_checks=True,
            skip_device_barrier=True,
        ),
        scratch_types=[
            pltpu.VMEM((C // 2,), jnp.int32),
            pltpu.VMEM((C,), jnp.float32),
            pltpu.VMEM((C,), jnp.float32),
            pltpu.VMEM((OCH,), jnp.float32),
            pltpu.VMEM((OCH,), jnp.float32),
            pltpu.SemaphoreType.DMA,
            pltpu.SemaphoreType.DMA,
            pltpu.SemaphoreType.DMA,
            pltpu.SemaphoreType.DMA,
            pltpu.SemaphoreType.DMA,
        ],
    )
    def _shuffle_rows(x_hbm, perm_hbm, out_hbm, perm_v, rin0, rin1, ob0, ob1,
                      sem_p, si0, si1, so0, so1):
        wid = lax.axis_index("s") * NC + lax.axis_index("c")
        r0 = wid * ROWS_PER_W
        rins = [rin0, rin1]
        in_sems = [si0, si1]
        obufs = [ob0, ob1]
        out_sems = [so0, so1]

        cp_perm = pltpu.async_copy(perm_hbm, perm_v, sem_p)
        cp_in = [pltpu.async_copy(x_hbm.at[r0], rin0, si0), None]
        cp_perm.wait()

        out_pending = [None, None]
        g = 0
        for rr in range(ROWS_PER_W):
            cur = rr % 2
            cp_in[cur].wait()
            if rr + 1 < ROWS_PER_W:
                nxt = (rr + 1) % 2
                cp_in[nxt] = pltpu.async_copy(
                    x_hbm.at[r0 + rr + 1], rins[nxt], in_sems[nxt])
            for k in range(NCHUNK):
                slot = g % 2
                if out_pending[slot] is not None:
                    out_pending[slot].wait()
                base = k * (OCH // 2)

                @plsc.parallel_loop(0, OCH // 2, step=L, unroll=4)
                def _(j, _base=base, _cur=cur, _slot=slot):
                    w = perm_v[pl.ds(_base + j, L)]
                    lo = lax.bitwise_and(w, jnp.int32(0xFFFF))
                    hi = lax.shift_right_logical(w, jnp.int32(16))
                    obufs[_slot][pl.ds(2 * j, L)] = plsc.load_gather(
                        rins[_cur], [lo])
                    obufs[_slot][pl.ds(2 * j + L, L)] = plsc.load_gather(
                        rins[_cur], [hi])

                out_pending[slot] = pltpu.async_copy(
                    obufs[slot], out_hbm.at[r0 + rr, pl.ds(k * OCH, OCH)],
                    out_sems[slot])
                g += 1
        for slot in (0, 1):
            if out_pending[slot] is not None:
                out_pending[slot].wait()

    return _shuffle_rows


def kernel(x):
    perm = jnp.asarray(_PERM_PACKED_HOST)
    return _build_shuffle()(x, perm)


# speedup vs baseline: 3.2412x; 1.0385x over previous
import functools

import numpy as np
import jax
import jax.numpy as jnp
from jax import lax
from jax.experimental import pallas as pl
from jax.experimental.pallas import tpu as pltpu
from jax.experimental.pallas import tpu_sc as plsc

R = 128
C = 32768
L = 16
NC = 2
NS = 16
NW = NC * NS
ROWS_PER_W = R // NW

_U32 = np.uint32


def _rotl(x, d):
    x = x.astype(np.uint32)
    return ((x << _U32(d)) | (x >> _U32(32 - d))).astype(np.uint32)


def _threefry2x32(k1, k2, x1, x2):
    k1 = _U32(k1)
    k2 = _U32(k2)
    ks = [k1, k2, _U32(k1 ^ k2 ^ _U32(0x1BD11BDA))]
    rot = [[13, 15, 26, 6], [17, 29, 16, 24]]
    x = np.asarray(x1, np.uint32) + ks[0]
    y = np.asarray(x2, np.uint32) + ks[1]

    def rounds(x, y, rs):
        for r in rs:
            x = (x + y).astype(np.uint32)
            y = _rotl(y, r)
            y = x ^ y
        return x, y

    for i, (rs, kx, kofs) in enumerate(
        [(rot[0], 1, 2), (rot[1], 2, 0), (rot[0], 0, 1),
         (rot[1], 1, 2), (rot[0], 2, 0)]
    ):
        x, y = rounds(x, y, rs)
        x = (x + ks[kx]).astype(np.uint32)
        y = (y + ks[kofs] + _U32(i + 1)).astype(np.uint32)
    return x, y


def _fixed_permutation(n=C, seed=42):
    a, b = _threefry2x32(_U32(0), _U32(seed), np.array([0], np.uint32),
                         np.array([0], np.uint32))
    key = (a[0], b[0])

    perm = np.arange(n, dtype=np.int32)
    num_rounds = int(np.ceil(3 * np.log(n) / np.log(2**32 - 1)))
    for _ in range(num_rounds):
        b1, b2 = _threefry2x32(key[0], key[1], np.zeros(2, np.uint32),
                               np.arange(2, dtype=np.uint32))
        key = (b1[0], b2[0])
        subkey = (b1[1], b2[1])
        c1, c2 = _threefry2x32(subkey[0], subkey[1], np.zeros(n, np.uint32),
                               np.arange(n, dtype=np.uint32))
        perm = perm[np.argsort((c1 ^ c2).astype(np.uint32), kind="stable")]
    return perm


_PERM_HOST = _fixed_permutation()

_PERM_PACKED_HOST = np.ascontiguousarray(
    _PERM_HOST.reshape(C // 32, 2, 16)[:, 0, :]
    | (_PERM_HOST.reshape(C // 32, 2, 16)[:, 1, :] << 16)
).reshape(C // 2).astype(np.int32)

OCH = 8192
NCHUNK = C // OCH


@functools.cache
def _build_shuffle():
    mesh = plsc.VectorSubcoreMesh(
        core_axis_name="c", subcore_axis_name="s", num_cores=NC, num_subcores=NS
    )

    @functools.partial(
        pl.kernel,
        out_type=jax.ShapeDtypeStruct((R, C), jnp.float32),
        mesh=mesh,
        compiler_params=pltpu.CompilerParams(
            needs_layout_passes=False,
            disable_bounds_checks=True,
            disable_semaphore_checks=True,
            skip_device_barrier=True,
        ),
        scratch_types=[
            pltpu.VMEM_SHARED((C // 2,), jnp.int32),
            pltpu.VMEM((C // 2,), jnp.int32),
            pltpu.VMEM((C,), jnp.float32),
            pltpu.VMEM((C,), jnp.float32),
            pltpu.VMEM((OCH,), jnp.float32),
            pltpu.VMEM((OCH,), jnp.float32),
            pltpu.SemaphoreType.DMA,
            pltpu.SemaphoreType.DMA,
            pltpu.SemaphoreType.DMA,
            pltpu.SemaphoreType.DMA,
            pltpu.SemaphoreType.DMA,
        ],
    )
    def _shuffle_rows(x_hbm, perm_hbm, out_hbm, perm_s, perm_v, rin0, rin1,
                      ob0, ob1, sem_p, si0, si1, so0, so1):
        sid = lax.axis_index("s")
        wid = sid * NC + lax.axis_index("c")
        r0 = wid * ROWS_PER_W
        rins = [rin0, rin1]
        in_sems = [si0, si1]
        obufs = [ob0, ob1]
        out_sems = [so0, so1]

        cp_in = [pltpu.async_copy(x_hbm.at[r0], rin0, si0), None]

        @pl.when(sid == 0)
        def _():
            pltpu.sync_copy(perm_hbm, perm_s)

        plsc.subcore_barrier()
        cp_perm = pltpu.async_copy(perm_s, perm_v, sem_p)
        cp_perm.wait()

        out_pending = [None, None]
        g = 0
        for rr in range(ROWS_PER_W):
            cur = rr % 2
            cp_in[cur].wait()
            if rr + 1 < ROWS_PER_W:
                nxt = (rr + 1) % 2
                cp_in[nxt] = pltpu.async_copy(
                    x_hbm.at[r0 + rr + 1], rins[nxt], in_sems[nxt])
            for k in range(NCHUNK):
                slot = g % 2
                if out_pending[slot] is not None:
                    out_pending[slot].wait()
                base = k * (OCH // 2)

                @plsc.parallel_loop(0, OCH // 2, step=L, unroll=4)
                def _(j, _base=base, _cur=cur, _slot=slot):
                    w = perm_v[pl.ds(_base + j, L)]
                    lo = lax.bitwise_and(w, jnp.int32(0xFFFF))
                    hi = lax.shift_right_logical(w, jnp.int32(16))
                    obufs[_slot][pl.ds(2 * j, L)] = plsc.load_gather(
                        rins[_cur], [lo])
                    obufs[_slot][pl.ds(2 * j + L, L)] = plsc.load_gather(
                        rins[_cur], [hi])

                out_pending[slot] = pltpu.async_copy(
                    obufs[slot], out_hbm.at[r0 + rr, pl.ds(k * OCH, OCH)],
                    out_sems[slot])
                g += 1
        for slot in (0, 1):
            if out_pending[slot] is not None:
                out_pending[slot].wait()

    return _shuffle_rows


def kernel(x):
    perm = jnp.asarray(_PERM_PACKED_HOST)
    return _build_shuffle()(x, perm)
